# baseline (speedup 1.0000x reference)
import dataclasses
import functools

import jax
import jax.numpy as jnp
from jax import lax
from jax.experimental import pallas as pl
from jax.experimental.pallas import tpu as pltpu
from jax.experimental.pallas import tpu_sc as plsc

_DIM = 128
_NC = 2
_NS = 16
_NW = _NC * _NS
_L = 16
_TC_CH = 512


def _sc_gather_fn(b):
    bw = b // _NW
    mesh = plsc.VectorSubcoreMesh(core_axis_name="c", subcore_axis_name="s")
    cp = pltpu.CompilerParams()
    if "needs_layout_passes" in pltpu.CompilerParams.__dataclass_fields__:
        cp = dataclasses.replace(cp, needs_layout_passes=False)

    @functools.partial(
        pl.kernel,
        out_type=[
            jax.ShapeDtypeStruct((4 * b, _DIM), jnp.float32),
            jax.ShapeDtypeStruct((2 * b, _DIM), jnp.float32),
        ],
        mesh=mesh,
        scratch_types=[
            pltpu.VMEM((3 * bw,), jnp.int32),
            pltpu.VMEM((3 * bw,), jnp.int32),
            pltpu.VMEM((6 * bw,), jnp.int32),
            pltpu.VMEM((6 * bw, _DIM), jnp.float32),
            pltpu.SemaphoreType.DMA,
            pltpu.SemaphoreType.DMA,
        ],
        compiler_params=cp,
    )
    def gather(ent_hbm, rel_hbm, pos_hbm, neg_hbm, oe_hbm, or_hbm,
               pos_v, neg_v, idx_v, rows_v, gsem, osem):
        wid = lax.axis_index("s") * _NC + lax.axis_index("c")
        base = wid * bw
        pltpu.sync_copy(pos_hbm.at[pl.ds(3 * base, 3 * bw)], pos_v)
        pltpu.sync_copy(neg_hbm.at[pl.ds(3 * base, 3 * bw)], neg_v)

        lane = lax.iota(jnp.int32, _L)
        for slot, (src, col) in enumerate(
                [(pos_v, 0), (pos_v, 2), (neg_v, 0), (neg_v, 2),
                 (pos_v, 1), (neg_v, 1)]):
            for j in range(bw // _L):
                pos = (lane + (j * _L)) * 3 + col
                idx_v[pl.ds(slot * bw + j * _L, _L)] = plsc.load_gather(
                    src, [pos])

        gathers = []
        for slot in range(4):
            gathers.append(pltpu.async_copy(
                ent_hbm.at[idx_v.at[pl.ds(slot * bw, bw)]],
                rows_v.at[pl.ds(slot * bw, bw)], gsem))
        for slot in range(4, 6):
            gathers.append(pltpu.async_copy(
                rel_hbm.at[idx_v.at[pl.ds(slot * bw, bw)]],
                rows_v.at[pl.ds(slot * bw, bw)], gsem))
        writes = []
        for slot in range(6):
            gathers[slot].wait()
            src = rows_v.at[pl.ds(slot * bw, bw)]
            if slot < 4:
                dst = oe_hbm.at[pl.ds(slot * b + base, bw)]
            else:
                dst = or_hbm.at[pl.ds((slot - 4) * b + base, bw)]
            writes.append(pltpu.async_copy(src, dst, osem))
        for wcopy in writes:
            wcopy.wait()

    return gather


def _unit(x):
    n = jnp.sqrt(jnp.sum(x * x, axis=1, keepdims=True))
    return x / jnp.maximum(n, 1e-12)


def _tc_loss_fn(inv_b):
    def _tc_loss(erows_ref, rrows_ref, out_ref):
        i = pl.program_id(0)
        hp = _unit(erows_ref[0])
        tp = _unit(erows_ref[1])
        hn = _unit(erows_ref[2])
        tn = _unit(erows_ref[3])
        rp = _unit(rrows_ref[0])
        rn = _unit(rrows_ref[1])
        dp = hp + rp - tp
        dn = hn + rn - tn
        ep = jnp.sqrt(jnp.sum(dp * dp, axis=1))
        en = jnp.sqrt(jnp.sum(dn * dn, axis=1))
        part = jnp.sum(jnp.maximum(1.0 + ep - en, 0.0))

        @pl.when(i == 0)
        def _():
            out_ref[...] = jnp.zeros((1, 1), jnp.float32)

        out_ref[...] += part.reshape(1, 1)

        @pl.when(i == pl.num_programs(0) - 1)
        def _():
            out_ref[...] *= inv_b

    return _tc_loss


@jax.jit
def kernel(pos_triples, neg_triples, ent_emb, rel_emb):
    b = pos_triples.shape[0]
    erows, rrows = _sc_gather_fn(b)(
        ent_emb, rel_emb,
        pos_triples.reshape(-1), neg_triples.reshape(-1))

    erows3 = erows.reshape(4, b, _DIM)
    rrows3 = rrows.reshape(2, b, _DIM)
    out = pl.pallas_call(
        _tc_loss_fn(1.0 / b),
        grid=(b // _TC_CH,),
        in_specs=[
            pl.BlockSpec((4, _TC_CH, _DIM), lambda i: (0, i, 0)),
            pl.BlockSpec((2, _TC_CH, _DIM), lambda i: (0, i, 0)),
        ],
        out_specs=pl.BlockSpec((1, 1), lambda i: (0, 0)),
        out_shape=jax.ShapeDtypeStruct((1, 1), jnp.float32),
    )(erows3, rrows3)
    return out[0, 0]
